# Initial kernel scaffold; baseline (speedup 1.0000x reference)
import jax
import jax.numpy as jnp
from jax.experimental import pallas as pl


def kernel(s_emb, t_emb, idx):
    raise NotImplementedError("write your pallas kernel here")


# speedup vs baseline: 9.2803x; 9.2803x over previous
import functools

import jax
import jax.numpy as jnp
from jax.experimental import pallas as pl
from jax.experimental.pallas import tpu as pltpu

N = 1024
D = 64
TOPK = 10
HALF = 5


def _loss_kernel(s_ref, t_ref, idxr_ref, idxc_ref, out_ref):
    f32 = jnp.float32
    s = s_ref[...]
    t = t_ref[...]

    s = s / jnp.maximum(jnp.sqrt(jnp.sum(s * s, axis=1, keepdims=True)), 1e-12)
    t = t / jnp.maximum(jnp.sqrt(jnp.sum(t * t, axis=1, keepdims=True)), 1e-12)

    s_sq = jnp.sum(s * s, axis=1, keepdims=True)
    gs = jax.lax.dot_general(s, s, (((1,), (1,)), ((), ())),
                             preferred_element_type=f32)
    d2_s = s_sq + jnp.transpose(s_sq) - 2.0 * gs
    S = jnp.sqrt(jnp.maximum(d2_s, 0.0))
    S = S / jnp.mean(S, axis=1, keepdims=True)

    t_sq = jnp.sum(t * t, axis=1, keepdims=True)
    gt = jax.lax.dot_general(t, t, (((1,), (1,)), ((), ())),
                             preferred_element_type=f32)
    d2_t = t_sq + jnp.transpose(t_sq) - 2.0 * gt
    W_P = jnp.exp(-jnp.maximum(d2_t, 0.0))

    same = jnp.broadcast_to(idxr_ref[...], (N, N)) == jnp.broadcast_to(
        idxc_ref[...], (N, N))
    A = jnp.where(same, 1.0, W_P)

    col = jax.lax.broadcasted_iota(jnp.int32, (N, N), 1)
    w_nn = jnp.zeros((N, N), dtype=f32)
    h_sum = jnp.zeros((N, N), dtype=f32)
    for k in range(TOPK):
        m = jnp.max(A, axis=1, keepdims=True)
        sel = jnp.min(jnp.where(A == m, col, N), axis=1, keepdims=True)
        onehot = (col == sel).astype(f32)
        w_nn = w_nn + onehot
        if k < HALF:
            h_sum = h_sum + onehot
        A = A - onehot * (A + 1.0)

    V = w_nn * jnp.transpose(w_nn)
    inner = jax.lax.dot_general(V, V, (((1,), (1,)), ((), ())),
                                preferred_element_type=f32)
    row_nnz = jnp.sum(V, axis=1, keepdims=True)
    W_C_tilda = V * inner / jnp.maximum(row_nnz, 1.0)
    W_C_hat = jax.lax.dot_general(h_sum, W_C_tilda, (((1,), (0,)), ((), ())),
                                  preferred_element_type=f32) * (1.0 / HALF)
    W_C = 0.5 * (W_C_hat + jnp.transpose(W_C_hat))
    W = 0.5 * (W_P + W_C)

    row = jax.lax.broadcasted_iota(jnp.int32, (N, N), 0)
    offdiag = (row != col).astype(f32)
    pull = jnp.maximum(S, 0.0) ** 2 * (W * offdiag)
    push = jnp.maximum(1.0 - S, 0.0) ** 2 * ((1.0 - W) * offdiag)
    loss = (jnp.sum(pull) + jnp.sum(push)) / (N * (N - 1))
    out_ref[...] = jnp.broadcast_to(loss, (1, 1))


@functools.partial(jax.jit, static_argnames=())
def _run(s_emb, t_emb, idx):
    idxf = idx.astype(jnp.float32)
    out = pl.pallas_call(
        _loss_kernel,
        out_shape=jax.ShapeDtypeStruct((1, 1), jnp.float32),
        compiler_params=pltpu.CompilerParams(
            vmem_limit_bytes=128 * 1024 * 1024),
    )(s_emb, t_emb, idxf.reshape(N, 1), idxf.reshape(1, N))
    return out[0, 0]


def kernel(s_emb, t_emb, idx):
    return _run(s_emb, t_emb, idx)
